# baseline (speedup 1.0000x reference)
import functools

import jax
import jax.numpy as jnp
from jax import lax
from jax.experimental import pallas as pl
from jax.experimental.pallas import tpu as pltpu
from jax.experimental.pallas import tpu_sc as plsc

_VOCAB = 100000
_D = 1024
_HALF = 512
_B = 16
_S = 512
_BASE = 10000.0
_N = _B * _S
_NC, _NS, _LANES = 2, 16, 16
_NW = _NC * _NS
_RPW = _N // _NW
_CH = 32
_NCH = _RPW // _CH
_NBUF = 3


def _rotate_chunk(buf, csv):
    def row(r, carry):
        for j in range(_HALF // _LANES):
            sl_e = pl.ds(j * _LANES, _LANES)
            sl_o = pl.ds(_HALF + j * _LANES, _LANES)
            e = buf[r, sl_e]
            o = buf[r, sl_o]
            c = csv[0, sl_e]
            s = csv[1, sl_e]
            buf[r, sl_e] = e * c - o * s
            buf[r, sl_o] = e * s + o * c
        return carry
    lax.fori_loop(0, _CH, row, 0)


def _body(table_hbm, idx_hbm, cs_hbm, out_hbm,
          idx_v, csv, b0, b1, b2, g0, g1, g2, w0, w1, w2):
    bufs = (b0, b1, b2)
    gsem = (g0, g1, g2)
    wsem = (w0, w1, w2)
    wid = lax.axis_index("s") * _NC + lax.axis_index("c")
    base = wid * _RPW
    pltpu.sync_copy(idx_hbm.at[wid], idx_v)
    pltpu.sync_copy(cs_hbm, csv)

    gcp = [None] * _NCH
    wcp = [None] * _NCH
    for i in range(_NBUF):
        gcp[i] = pltpu.async_copy(table_hbm.at[idx_v.at[i]], bufs[i], gsem[i])
    for i in range(_NCH):
        b = i % _NBUF
        gcp[i].wait()
        _rotate_chunk(bufs[b], csv)
        wcp[i] = pltpu.async_copy(
            bufs[b], out_hbm.at[pl.ds(base + i * _CH, _CH)], wsem[b])
        nxt = i + 2
        if i >= 1 and nxt < _NCH:
            wcp[i - 1].wait()
            gcp[nxt] = pltpu.async_copy(
                table_hbm.at[idx_v.at[nxt]], bufs[nxt % _NBUF],
                gsem[nxt % _NBUF])
    for i in range(_NCH - _NBUF, _NCH):
        wcp[i].wait()


@functools.cache
def _sc_lookup():
    return pl.kernel(
        _body,
        mesh=plsc.VectorSubcoreMesh(core_axis_name="c", subcore_axis_name="s"),
        out_type=jax.ShapeDtypeStruct((_N, _D), jnp.float32),
        scratch_types=[
            pltpu.VMEM((_NCH, _CH), jnp.int32),
            pltpu.VMEM((2, _HALF), jnp.float32),
            pltpu.VMEM((_CH, _D), jnp.float32),
            pltpu.VMEM((_CH, _D), jnp.float32),
            pltpu.VMEM((_CH, _D), jnp.float32),
            pltpu.SemaphoreType.DMA,
            pltpu.SemaphoreType.DMA,
            pltpu.SemaphoreType.DMA,
            pltpu.SemaphoreType.DMA,
            pltpu.SemaphoreType.DMA,
            pltpu.SemaphoreType.DMA,
        ],
    )


def _cos_sin():
    fi = jnp.arange(_HALF, dtype=jnp.float32)
    freqs = 1.0 / (_BASE ** (fi / _D))
    pos = jnp.arange(_S, dtype=jnp.float32)
    ang = pos * freqs
    return jnp.stack([jnp.cos(ang), jnp.sin(ang)])


def kernel(x, table):
    idx = x.reshape(_NW, _NCH, _CH).astype(jnp.int32)
    cs = _cos_sin()
    out = _sc_lookup()(table, idx, cs)
    return out.reshape(_B, _S, _D)


# speedup vs baseline: 1.8577x; 1.8577x over previous
import functools

import jax
import jax.numpy as jnp
from jax import lax
from jax.experimental import pallas as pl
from jax.experimental.pallas import tpu as pltpu
from jax.experimental.pallas import tpu_sc as plsc

_VOCAB = 100000
_D = 1024
_HALF = 512
_B = 16
_S = 512
_BASE = 10000.0
_N = _B * _S
_NC, _NS, _LANES = 2, 16, 16
_NW = _NC * _NS
_RPW = _N // _NW
_CH = 32
_NCH = _RPW // _CH
_NBUF = 3


def _rotate_chunk(buf, csv):
    def group(j, carry):
        jb = j * _LANES
        sl_e = pl.ds(jb, _LANES)
        sl_o = pl.ds(_HALF + jb, _LANES)
        c = csv[0, sl_e]
        s = csv[1, sl_e]

        @plsc.parallel_loop(0, _CH, unroll=4)
        def _row(r):
            e = buf[r, sl_e]
            o = buf[r, sl_o]
            buf[r, sl_e] = e * c - o * s
            buf[r, sl_o] = e * s + o * c

        return carry

    lax.fori_loop(0, _HALF // _LANES, group, 0)


def _body(table_hbm, idx_hbm, cs_hbm, out_hbm,
          idx_v, csv, b0, b1, b2, g0, g1, g2, w0, w1, w2):
    bufs = (b0, b1, b2)
    gsem = (g0, g1, g2)
    wsem = (w0, w1, w2)
    wid = lax.axis_index("s") * _NC + lax.axis_index("c")
    base = wid * _RPW
    pltpu.sync_copy(idx_hbm.at[wid], idx_v)
    pltpu.sync_copy(cs_hbm, csv)

    gcp = [None] * _NCH
    wcp = [None] * _NCH
    for i in range(_NBUF):
        gcp[i] = pltpu.async_copy(table_hbm.at[idx_v.at[i]], bufs[i], gsem[i])
    for i in range(_NCH):
        b = i % _NBUF
        gcp[i].wait()
        _rotate_chunk(bufs[b], csv)
        wcp[i] = pltpu.async_copy(
            bufs[b], out_hbm.at[pl.ds(base + i * _CH, _CH)], wsem[b])
        nxt = i + 2
        if i >= 1 and nxt < _NCH:
            wcp[i - 1].wait()
            gcp[nxt] = pltpu.async_copy(
                table_hbm.at[idx_v.at[nxt]], bufs[nxt % _NBUF],
                gsem[nxt % _NBUF])
    for i in range(_NCH - _NBUF, _NCH):
        wcp[i].wait()


@functools.cache
def _sc_lookup():
    return pl.kernel(
        _body,
        mesh=plsc.VectorSubcoreMesh(core_axis_name="c", subcore_axis_name="s"),
        out_type=jax.ShapeDtypeStruct((_N, _D), jnp.float32),
        scratch_types=[
            pltpu.VMEM((_NCH, _CH), jnp.int32),
            pltpu.VMEM((2, _HALF), jnp.float32),
            pltpu.VMEM((_CH, _D), jnp.float32),
            pltpu.VMEM((_CH, _D), jnp.float32),
            pltpu.VMEM((_CH, _D), jnp.float32),
            pltpu.SemaphoreType.DMA,
            pltpu.SemaphoreType.DMA,
            pltpu.SemaphoreType.DMA,
            pltpu.SemaphoreType.DMA,
            pltpu.SemaphoreType.DMA,
            pltpu.SemaphoreType.DMA,
        ],
    )


def _cos_sin():
    fi = jnp.arange(_HALF, dtype=jnp.float32)
    freqs = 1.0 / (_BASE ** (fi / _D))
    pos = jnp.arange(_S, dtype=jnp.float32)
    ang = pos * freqs
    return jnp.stack([jnp.cos(ang), jnp.sin(ang)])


def kernel(x, table):
    idx = x.reshape(_NW, _NCH, _CH).astype(jnp.int32)
    cs = _cos_sin()
    out = _sc_lookup()(table, idx, cs)
    return out.reshape(_B, _S, _D)
